# Initial kernel scaffold; baseline (speedup 1.0000x reference)
import jax
import jax.numpy as jnp
from jax.experimental import pallas as pl


def kernel(user_idx, user_norm_price, item_idx, item_cat, item_color, item_graphic, item_norm_price, user_table, item_table, cat_table, color_table, graphic_table, uW1, ub1, uW2, ub2, iW1, ib1, iW2, ib2):
    raise NotImplementedError("write your pallas kernel here")


# speedup vs baseline: 1.9803x; 1.9803x over previous
import functools

import jax
import jax.numpy as jnp
from jax import lax
from jax.experimental import pallas as pl
from jax.experimental.pallas import tpu as pltpu
from jax.experimental.pallas import tpu_sc as plsc

_B = 16384
_EMB = 64
_TAG = 32
_HID = 256
_OUT = 128

_NC = 2
_NS = 16
_NW = _NC * _NS
_BPW = _B // _NW

_BT = 1024
_F32 = jnp.float32


def _sc_gather_body(uidx, iidx, cidx, clidx, gidx,
                    utab, itab, ctab, cltab, gtab,
                    ue_o, ie_o, ce_o, cle_o, ge_o,
                    uix_v, iix_v, cix_v, clix_v, gix_v,
                    ue_v, ie_v, ce_v, cle_v, ge_v, sem):
    wid = lax.axis_index("s") * _NC + lax.axis_index("c")
    base = wid * _BPW
    sl = pl.ds(base, _BPW)
    pltpu.sync_copy(uidx.at[sl], uix_v)
    pltpu.sync_copy(iidx.at[sl], iix_v)
    pltpu.sync_copy(cidx.at[sl], cix_v)
    pltpu.sync_copy(clidx.at[sl], clix_v)
    pltpu.sync_copy(gidx.at[sl], gix_v)
    c0 = pltpu.async_copy(utab.at[uix_v], ue_v, sem)
    c1 = pltpu.async_copy(itab.at[iix_v], ie_v, sem)
    c2 = pltpu.async_copy(ctab.at[cix_v], ce_v, sem)
    c3 = pltpu.async_copy(cltab.at[clix_v], cle_v, sem)
    c4 = pltpu.async_copy(gtab.at[gix_v], ge_v, sem)
    c0.wait()
    pltpu.sync_copy(ue_v, ue_o.at[sl])
    c1.wait()
    pltpu.sync_copy(ie_v, ie_o.at[sl])
    c2.wait()
    pltpu.sync_copy(ce_v, ce_o.at[sl])
    c3.wait()
    pltpu.sync_copy(cle_v, cle_o.at[sl])
    c4.wait()
    pltpu.sync_copy(ge_v, ge_o.at[sl])


@functools.cache
def _sc_gather():
    return pl.kernel(
        _sc_gather_body,
        mesh=plsc.VectorSubcoreMesh(core_axis_name="c", subcore_axis_name="s"),
        compiler_params=pltpu.CompilerParams(use_tc_tiling_on_sc=False),
        out_type=[
            jax.ShapeDtypeStruct((_B, _EMB), _F32),
            jax.ShapeDtypeStruct((_B, _EMB), _F32),
            jax.ShapeDtypeStruct((_B, _TAG), _F32),
            jax.ShapeDtypeStruct((_B, _TAG), _F32),
            jax.ShapeDtypeStruct((_B, _TAG), _F32),
        ],
        scratch_types=[
            pltpu.VMEM((_BPW,), jnp.int32),
            pltpu.VMEM((_BPW,), jnp.int32),
            pltpu.VMEM((_BPW,), jnp.int32),
            pltpu.VMEM((_BPW,), jnp.int32),
            pltpu.VMEM((_BPW,), jnp.int32),
            pltpu.VMEM((_BPW, _EMB), _F32),
            pltpu.VMEM((_BPW, _EMB), _F32),
            pltpu.VMEM((_BPW, _TAG), _F32),
            pltpu.VMEM((_BPW, _TAG), _F32),
            pltpu.VMEM((_BPW, _TAG), _F32),
            pltpu.SemaphoreType.DMA,
        ],
    )


def _tc_towers_body(ue, uprice, ie, ce, cle, ge, iprice,
                    uW1a, uW1p, ub1, uW2, ub2,
                    iW1a, iW1b, iW1c, iW1d, iW1p, ib1, iW2, ib2,
                    out):
    uh = jnp.dot(ue[...], uW1a[...], preferred_element_type=_F32)
    uh = uh + uprice[...][:, None] * uW1p[...] + ub1[...]
    uh = jnp.maximum(uh, 0.0)
    uvec = jnp.dot(uh, uW2[...], preferred_element_type=_F32) + ub2[...]

    ih = jnp.dot(ie[...], iW1a[...], preferred_element_type=_F32)
    ih = ih + jnp.dot(ce[...], iW1b[...], preferred_element_type=_F32)
    ih = ih + jnp.dot(cle[...], iW1c[...], preferred_element_type=_F32)
    ih = ih + jnp.dot(ge[...], iW1d[...], preferred_element_type=_F32)
    ih = ih + iprice[...][:, None] * iW1p[...] + ib1[...]
    ih = jnp.maximum(ih, 0.0)
    ivec = jnp.dot(ih, iW2[...], preferred_element_type=_F32) + ib2[...]

    un = jnp.sqrt(jnp.sum(uvec * uvec, axis=1))
    inrm = jnp.sqrt(jnp.sum(ivec * ivec, axis=1))
    denom = jnp.maximum(un, 1e-12) * jnp.maximum(inrm, 1e-12)
    out[...] = jnp.sum(uvec * ivec, axis=1) / denom


def _row_spec(cols):
    return pl.BlockSpec((_BT, cols), lambda i: (i, 0))


def _full_spec(r, c):
    return pl.BlockSpec((r, c), lambda i: (0, 0))


_tc_towers = pl.pallas_call(
    _tc_towers_body,
    grid=(_B // _BT,),
    in_specs=[
        _row_spec(_EMB),
        pl.BlockSpec((_BT,), lambda i: (i,)),
        _row_spec(_EMB),
        _row_spec(_TAG),
        _row_spec(_TAG),
        _row_spec(_TAG),
        pl.BlockSpec((_BT,), lambda i: (i,)),
        _full_spec(_EMB, _HID),
        _full_spec(1, _HID),
        _full_spec(1, _HID),
        _full_spec(_HID, _OUT),
        _full_spec(1, _OUT),
        _full_spec(_EMB, _HID),
        _full_spec(_TAG, _HID),
        _full_spec(_TAG, _HID),
        _full_spec(_TAG, _HID),
        _full_spec(1, _HID),
        _full_spec(1, _HID),
        _full_spec(_HID, _OUT),
        _full_spec(1, _OUT),
    ],
    out_specs=pl.BlockSpec((_BT,), lambda i: (i,)),
    out_shape=jax.ShapeDtypeStruct((_B,), _F32),
)


def kernel(user_idx, user_norm_price, item_idx, item_cat, item_color,
           item_graphic, item_norm_price, user_table, item_table, cat_table,
           color_table, graphic_table, uW1, ub1, uW2, ub2, iW1, ib1, iW2, ib2):
    i32 = jnp.int32
    ue, ie, ce, cle, ge = _sc_gather()(
        user_idx.astype(i32), item_idx.astype(i32), item_cat.astype(i32),
        item_color.astype(i32), item_graphic.astype(i32),
        user_table, item_table, cat_table, color_table, graphic_table)
    return _tc_towers(
        ue, user_norm_price, ie, ce, cle, ge, item_norm_price,
        uW1[:_EMB], uW1[_EMB:], ub1[None, :], uW2, ub2[None, :],
        iW1[:_EMB], iW1[_EMB:_EMB + _TAG], iW1[_EMB + _TAG:_EMB + 2 * _TAG],
        iW1[_EMB + 2 * _TAG:_EMB + 3 * _TAG], iW1[_EMB + 3 * _TAG:],
        ib1[None, :], iW2, ib2[None, :])


# speedup vs baseline: 2.4354x; 1.2298x over previous
import functools

import jax
import jax.numpy as jnp
from jax import lax
from jax.experimental import pallas as pl
from jax.experimental.pallas import tpu as pltpu
from jax.experimental.pallas import tpu_sc as plsc

_B = 16384
_EMB = 64
_TAG = 32
_HID = 256
_OUT = 128

_NC = 2
_NS = 16
_NW = _NC * _NS
_BPW = _B // _NW
_TCH = 128
_TROWS = 256

_BT = 1024
_F32 = jnp.float32


def _sc_gather_body(uidx, iidx, cidx, clidx, gidx,
                    utab, itab, ctab, cltab, gtab,
                    ue_o, ie_o, ce_o, cle_o, ge_o,
                    idx_v, big_v, tag_v, sem, semt):
    wid = lax.axis_index("s") * _NC + lax.axis_index("c")
    base = wid * _BPW
    sl = pl.ds(base, _BPW)

    def gather_big(tab, idx_hbm, out_hbm):
        pltpu.sync_copy(idx_hbm.at[sl], idx_v)

        def row16(j, _):
            v = idx_v[pl.ds(j * 16, 16)]
            for k in range(16):
                pltpu.async_copy(tab.at[pl.ds(v[k], 1)],
                                 big_v.at[pl.ds(j * 16 + k, 1)], sem)
            return _

        lax.fori_loop(0, _BPW // 16, row16, 0)
        pltpu.make_async_copy(tab.at[pl.ds(0, _BPW)], big_v, sem).wait()
        pltpu.sync_copy(big_v, out_hbm.at[sl])

    def gather_tag(tab, idx_hbm, out_hbm):
        pltpu.sync_copy(idx_hbm.at[sl], idx_v)
        for r in range(_BPW // _TROWS):
            for h in range(_TROWS // _TCH):
                o = r * _TROWS + h * _TCH
                pltpu.async_copy(
                    tab.at[idx_v.at[pl.ds(o, _TCH)]],
                    tag_v.at[pl.ds(h * _TCH, _TCH)], semt)
            pltpu.make_async_copy(tab.at[pl.ds(0, _TROWS)], tag_v,
                                  semt).wait()
            pltpu.sync_copy(tag_v,
                            out_hbm.at[pl.ds(base + r * _TROWS, _TROWS)])

    gather_big(utab, uidx, ue_o)
    gather_big(itab, iidx, ie_o)
    gather_tag(ctab, cidx, ce_o)
    gather_tag(cltab, clidx, cle_o)
    gather_tag(gtab, gidx, ge_o)


@functools.cache
def _sc_gather():
    return pl.kernel(
        _sc_gather_body,
        mesh=plsc.VectorSubcoreMesh(core_axis_name="c", subcore_axis_name="s"),
        out_type=[
            jax.ShapeDtypeStruct((_B, _EMB), _F32),
            jax.ShapeDtypeStruct((_B, _EMB), _F32),
            jax.ShapeDtypeStruct((_B, 128), _F32),
            jax.ShapeDtypeStruct((_B, 128), _F32),
            jax.ShapeDtypeStruct((_B, 128), _F32),
        ],
        scratch_types=[
            pltpu.VMEM((_BPW,), jnp.int32),
            pltpu.VMEM((_BPW, _EMB), _F32),
            pltpu.VMEM((_TROWS, 128), _F32),
            pltpu.SemaphoreType.DMA,
            pltpu.SemaphoreType.DMA,
        ],
    )


def _tc_towers_body(ue, uprice, ie, ce, cle, ge, iprice,
                    uW1a, uW1p, ub1, uW2, ub2,
                    iW1a, iW1b, iW1c, iW1d, iW1p, ib1, iW2, ib2,
                    out):
    uh = jnp.dot(ue[...], uW1a[...], preferred_element_type=_F32)
    uh = uh + uprice[...][:, None] * uW1p[...] + ub1[...]
    uh = jnp.maximum(uh, 0.0)
    uvec = jnp.dot(uh, uW2[...], preferred_element_type=_F32) + ub2[...]

    ih = jnp.dot(ie[...], iW1a[...], preferred_element_type=_F32)
    ih = ih + jnp.dot(ce[:, :_TAG], iW1b[...], preferred_element_type=_F32)
    ih = ih + jnp.dot(cle[:, :_TAG], iW1c[...], preferred_element_type=_F32)
    ih = ih + jnp.dot(ge[:, :_TAG], iW1d[...], preferred_element_type=_F32)
    ih = ih + iprice[...][:, None] * iW1p[...] + ib1[...]
    ih = jnp.maximum(ih, 0.0)
    ivec = jnp.dot(ih, iW2[...], preferred_element_type=_F32) + ib2[...]

    un = jnp.sqrt(jnp.sum(uvec * uvec, axis=1))
    inrm = jnp.sqrt(jnp.sum(ivec * ivec, axis=1))
    denom = jnp.maximum(un, 1e-12) * jnp.maximum(inrm, 1e-12)
    out[...] = jnp.sum(uvec * ivec, axis=1) / denom


def _row_spec(cols):
    return pl.BlockSpec((_BT, cols), lambda i: (i, 0))


def _full_spec(r, c):
    return pl.BlockSpec((r, c), lambda i: (0, 0))


_tc_towers = pl.pallas_call(
    _tc_towers_body,
    grid=(_B // _BT,),
    in_specs=[
        _row_spec(_EMB),
        pl.BlockSpec((_BT,), lambda i: (i,)),
        _row_spec(_EMB),
        _row_spec(128),
        _row_spec(128),
        _row_spec(128),
        pl.BlockSpec((_BT,), lambda i: (i,)),
        _full_spec(_EMB, _HID),
        _full_spec(1, _HID),
        _full_spec(1, _HID),
        _full_spec(_HID, _OUT),
        _full_spec(1, _OUT),
        _full_spec(_EMB, _HID),
        _full_spec(_TAG, _HID),
        _full_spec(_TAG, _HID),
        _full_spec(_TAG, _HID),
        _full_spec(1, _HID),
        _full_spec(1, _HID),
        _full_spec(_HID, _OUT),
        _full_spec(1, _OUT),
    ],
    out_specs=pl.BlockSpec((_BT,), lambda i: (i,)),
    out_shape=jax.ShapeDtypeStruct((_B,), _F32),
)


def kernel(user_idx, user_norm_price, item_idx, item_cat, item_color,
           item_graphic, item_norm_price, user_table, item_table, cat_table,
           color_table, graphic_table, uW1, ub1, uW2, ub2, iW1, ib1, iW2, ib2):
    i32 = jnp.int32
    pad = ((0, 0), (0, 128 - _TAG))
    ue, ie, ce, cle, ge = _sc_gather()(
        user_idx.astype(i32), item_idx.astype(i32), item_cat.astype(i32),
        item_color.astype(i32), item_graphic.astype(i32),
        user_table, item_table,
        jnp.pad(cat_table, pad), jnp.pad(color_table, pad),
        jnp.pad(graphic_table, pad))
    return _tc_towers(
        ue, user_norm_price, ie, ce, cle, ge, item_norm_price,
        uW1[:_EMB], uW1[_EMB:], ub1[None, :], uW2, ub2[None, :],
        iW1[:_EMB], iW1[_EMB:_EMB + _TAG], iW1[_EMB + _TAG:_EMB + 2 * _TAG],
        iW1[_EMB + 2 * _TAG:_EMB + 3 * _TAG], iW1[_EMB + 3 * _TAG:],
        ib1[None, :], iW2, ib2[None, :])


# speedup vs baseline: 2.6621x; 1.0931x over previous
import functools

import jax
import jax.numpy as jnp
from jax import lax
from jax.experimental import pallas as pl
from jax.experimental.pallas import tpu as pltpu
from jax.experimental.pallas import tpu_sc as plsc

_B = 16384
_EMB = 64
_TAG = 32
_HID = 256
_OUT = 128

_NC = 2
_NS = 16
_NW = _NC * _NS
_BPW = _B // _NW
_TCH = 128
_TROWS = 256

_BT = 1024
_F32 = jnp.float32


def _wid_base():
    wid = lax.axis_index("s") * _NC + lax.axis_index("c")
    return wid * _BPW


def _gather_big(tab, idx_hbm, out_hbm, idx_v, big_v, sem, base):
    sl = pl.ds(base, _BPW)
    pltpu.sync_copy(idx_hbm.at[sl], idx_v)

    def row16(j, _):
        v = idx_v[pl.ds(j * 16, 16)]
        for k in range(16):
            pltpu.async_copy(tab.at[pl.ds(v[k], 1)],
                             big_v.at[pl.ds(j * 16 + k, 1)], sem)
        return _

    lax.fori_loop(0, _BPW // 16, row16, 0)
    pltpu.make_async_copy(tab.at[pl.ds(0, _BPW)], big_v, sem).wait()
    pltpu.sync_copy(big_v, out_hbm.at[sl])


def _gather_tag(tab, idx_hbm, out_hbm, idx_v, tag_v, semt, base):
    sl = pl.ds(base, _BPW)
    pltpu.sync_copy(idx_hbm.at[sl], idx_v)
    for r in range(_BPW // _TROWS):
        for h in range(_TROWS // _TCH):
            o = r * _TROWS + h * _TCH
            pltpu.async_copy(
                tab.at[idx_v.at[pl.ds(o, _TCH)]],
                tag_v.at[pl.ds(h * _TCH, _TCH)], semt)
        pltpu.make_async_copy(tab.at[pl.ds(0, _TROWS)], tag_v, semt).wait()
        pltpu.sync_copy(tag_v, out_hbm.at[pl.ds(base + r * _TROWS, _TROWS)])


def _sc_user_tags_body(uidx, cidx, clidx, gidx, utab, ctab, cltab, gtab,
                       ue_o, ce_o, cle_o, ge_o,
                       idx_v, big_v, tag_v, sem, semt):
    base = _wid_base()
    _gather_big(utab, uidx, ue_o, idx_v, big_v, sem, base)
    _gather_tag(ctab, cidx, ce_o, idx_v, tag_v, semt, base)
    _gather_tag(cltab, clidx, cle_o, idx_v, tag_v, semt, base)
    _gather_tag(gtab, gidx, ge_o, idx_v, tag_v, semt, base)


def _sc_item_body(iidx, itab, ie_o, idx_v, big_v, sem):
    base = _wid_base()
    _gather_big(itab, iidx, ie_o, idx_v, big_v, sem, base)


@functools.cache
def _sc_user_tags():
    return pl.kernel(
        _sc_user_tags_body,
        mesh=plsc.VectorSubcoreMesh(core_axis_name="c", subcore_axis_name="s"),
        out_type=[
            jax.ShapeDtypeStruct((_B, _EMB), _F32),
            jax.ShapeDtypeStruct((_B, 128), _F32),
            jax.ShapeDtypeStruct((_B, 128), _F32),
            jax.ShapeDtypeStruct((_B, 128), _F32),
        ],
        scratch_types=[
            pltpu.VMEM((_BPW,), jnp.int32),
            pltpu.VMEM((_BPW, _EMB), _F32),
            pltpu.VMEM((_TROWS, 128), _F32),
            pltpu.SemaphoreType.DMA,
            pltpu.SemaphoreType.DMA,
        ],
    )


@functools.cache
def _sc_item():
    return pl.kernel(
        _sc_item_body,
        mesh=plsc.VectorSubcoreMesh(core_axis_name="c", subcore_axis_name="s"),
        out_type=jax.ShapeDtypeStruct((_B, _EMB), _F32),
        scratch_types=[
            pltpu.VMEM((_BPW,), jnp.int32),
            pltpu.VMEM((_BPW, _EMB), _F32),
            pltpu.SemaphoreType.DMA,
        ],
    )


def _tc_towers_body(ue, uprice, ie, ce, cle, ge, iprice,
                    uW1a, uW1p, ub1, uW2, ub2,
                    iW1a, iW1b, iW1c, iW1d, iW1p, ib1, iW2, ib2,
                    out):
    uh = jnp.dot(ue[...], uW1a[...], preferred_element_type=_F32)
    uh = uh + uprice[...][:, None] * uW1p[...] + ub1[...]
    uh = jnp.maximum(uh, 0.0)
    uvec = jnp.dot(uh, uW2[...], preferred_element_type=_F32) + ub2[...]

    ih = jnp.dot(ie[...], iW1a[...], preferred_element_type=_F32)
    ih = ih + jnp.dot(ce[:, :_TAG], iW1b[...], preferred_element_type=_F32)
    ih = ih + jnp.dot(cle[:, :_TAG], iW1c[...], preferred_element_type=_F32)
    ih = ih + jnp.dot(ge[:, :_TAG], iW1d[...], preferred_element_type=_F32)
    ih = ih + iprice[...][:, None] * iW1p[...] + ib1[...]
    ih = jnp.maximum(ih, 0.0)
    ivec = jnp.dot(ih, iW2[...], preferred_element_type=_F32) + ib2[...]

    un = jnp.sqrt(jnp.sum(uvec * uvec, axis=1))
    inrm = jnp.sqrt(jnp.sum(ivec * ivec, axis=1))
    denom = jnp.maximum(un, 1e-12) * jnp.maximum(inrm, 1e-12)
    out[...] = jnp.sum(uvec * ivec, axis=1) / denom


def _row_spec(cols):
    return pl.BlockSpec((_BT, cols), lambda i: (i, 0))


def _full_spec(r, c):
    return pl.BlockSpec((r, c), lambda i: (0, 0))


_tc_towers = pl.pallas_call(
    _tc_towers_body,
    grid=(_B // _BT,),
    in_specs=[
        _row_spec(_EMB),
        pl.BlockSpec((_BT,), lambda i: (i,)),
        _row_spec(_EMB),
        _row_spec(128),
        _row_spec(128),
        _row_spec(128),
        pl.BlockSpec((_BT,), lambda i: (i,)),
        _full_spec(_EMB, _HID),
        _full_spec(1, _HID),
        _full_spec(1, _HID),
        _full_spec(_HID, _OUT),
        _full_spec(1, _OUT),
        _full_spec(_EMB, _HID),
        _full_spec(_TAG, _HID),
        _full_spec(_TAG, _HID),
        _full_spec(_TAG, _HID),
        _full_spec(1, _HID),
        _full_spec(1, _HID),
        _full_spec(_HID, _OUT),
        _full_spec(1, _OUT),
    ],
    out_specs=pl.BlockSpec((_BT,), lambda i: (i,)),
    out_shape=jax.ShapeDtypeStruct((_B,), _F32),
)


def kernel(user_idx, user_norm_price, item_idx, item_cat, item_color,
           item_graphic, item_norm_price, user_table, item_table, cat_table,
           color_table, graphic_table, uW1, ub1, uW2, ub2, iW1, ib1, iW2, ib2):
    i32 = jnp.int32
    pad = ((0, 0), (0, 128 - _TAG))
    ie = _sc_item()(item_idx.astype(i32), item_table)
    ue, ce, cle, ge = _sc_user_tags()(
        user_idx.astype(i32), item_cat.astype(i32),
        item_color.astype(i32), item_graphic.astype(i32),
        user_table,
        jnp.pad(cat_table, pad), jnp.pad(color_table, pad),
        jnp.pad(graphic_table, pad))
    return _tc_towers(
        ue, user_norm_price, ie, ce, cle, ge, item_norm_price,
        uW1[:_EMB], uW1[_EMB:], ub1[None, :], uW2, ub2[None, :],
        iW1[:_EMB], iW1[_EMB:_EMB + _TAG], iW1[_EMB + _TAG:_EMB + 2 * _TAG],
        iW1[_EMB + 2 * _TAG:_EMB + 3 * _TAG], iW1[_EMB + 3 * _TAG:],
        ib1[None, :], iW2, ib2[None, :])


# speedup vs baseline: 2.8377x; 1.0660x over previous
import functools

import jax
import jax.numpy as jnp
from jax import lax
from jax.experimental import pallas as pl
from jax.experimental.pallas import tpu as pltpu
from jax.experimental.pallas import tpu_sc as plsc

_B = 16384
_EMB = 64
_TAG = 32
_HID = 256
_OUT = 128

_NC = 2
_NS = 16
_NW = _NC * _NS
_BPW = _B // _NW
_TCH = 128

_BT = 1024
_F32 = jnp.float32


def _wid_base():
    wid = lax.axis_index("s") * _NC + lax.axis_index("c")
    return wid * _BPW


def _gather_big(tab, idx_hbm, out_hbm, idx_v, big_v, sem, base):
    rows = big_v.shape[0]
    pltpu.sync_copy(idx_hbm.at[pl.ds(base, _BPW)], idx_v)
    for r in range(_BPW // rows):

        def row16(j, _, r=r):
            v = idx_v[pl.ds(r * rows + j * 16, 16)]
            for k in range(16):
                pltpu.async_copy(tab.at[pl.ds(v[k], 1)],
                                 big_v.at[pl.ds(j * 16 + k, 1)], sem)
            return _

        lax.fori_loop(0, rows // 16, row16, 0)
        pltpu.make_async_copy(tab.at[pl.ds(0, rows)], big_v, sem).wait()
        pltpu.sync_copy(big_v, out_hbm.at[pl.ds(base + r * rows, rows)])


def _gather_tag(tab, idx_hbm, out_hbm, idx_v, tag_v, semt, base):
    sl = pl.ds(base, _BPW)
    pltpu.sync_copy(idx_hbm.at[sl], idx_v)
    for h in range(_BPW // _TCH):
        pltpu.async_copy(
            tab.at[idx_v.at[pl.ds(h * _TCH, _TCH)]],
            tag_v.at[pl.ds(h * _TCH, _TCH)], semt)
    pltpu.make_async_copy(tab.at[pl.ds(0, _BPW)], tag_v, semt).wait()
    pltpu.sync_copy(tag_v, out_hbm.at[sl])


def _sc_user_tags_body(uidx, cidx, clidx, gidx, utab, ctab, cltab, gtab,
                       ue_o, ce_o, cle_o, ge_o,
                       idx_v, big_v, tag_v, sem, semt):
    base = _wid_base()
    _gather_big(utab, uidx, ue_o, idx_v, big_v, sem, base)
    _gather_tag(ctab, cidx, ce_o, idx_v, tag_v, semt, base)
    _gather_tag(cltab, clidx, cle_o, idx_v, tag_v, semt, base)
    _gather_tag(gtab, gidx, ge_o, idx_v, tag_v, semt, base)


def _sc_item_body(iidx, itab, ie_o, idx_v, big_v, sem):
    base = _wid_base()
    _gather_big(itab, iidx, ie_o, idx_v, big_v, sem, base)


@functools.cache
def _sc_user_tags():
    return pl.kernel(
        _sc_user_tags_body,
        mesh=plsc.VectorSubcoreMesh(core_axis_name="c", subcore_axis_name="s"),
        out_type=[
            jax.ShapeDtypeStruct((_B, _EMB), _F32),
            jax.ShapeDtypeStruct((_B, 128), _F32),
            jax.ShapeDtypeStruct((_B, 128), _F32),
            jax.ShapeDtypeStruct((_B, 128), _F32),
        ],
        scratch_types=[
            pltpu.VMEM((_BPW,), jnp.int32),
            pltpu.VMEM((_BPW // 2, _EMB), _F32),
            pltpu.VMEM((_BPW, 128), _F32),
            pltpu.SemaphoreType.DMA,
            pltpu.SemaphoreType.DMA,
        ],
    )


@functools.cache
def _sc_item():
    return pl.kernel(
        _sc_item_body,
        mesh=plsc.VectorSubcoreMesh(core_axis_name="c", subcore_axis_name="s"),
        out_type=jax.ShapeDtypeStruct((_B, _EMB), _F32),
        scratch_types=[
            pltpu.VMEM((_BPW,), jnp.int32),
            pltpu.VMEM((_BPW, _EMB), _F32),
            pltpu.SemaphoreType.DMA,
        ],
    )


_K1 = _EMB + _EMB + 3 * _TAG
_H2 = 2 * _HID


def _tc_towers_body(ue, ie, ce, cle, ge, uprice, iprice,
                    W1c, uW1p, iW1p, b1c, W2c, b2c, out):
    x = jnp.concatenate(
        [ue[...], ie[...], ce[:, :_TAG], cle[:, :_TAG], ge[:, :_TAG]], axis=1)
    h = jnp.dot(x, W1c[...], preferred_element_type=_F32) + b1c[...]
    hu = h[:, :_HID] + uprice[...][:, None] * uW1p[...]
    hi = h[:, _HID:] + iprice[...][:, None] * iW1p[...]
    h = jnp.concatenate([jnp.maximum(hu, 0.0), jnp.maximum(hi, 0.0)], axis=1)
    y = jnp.dot(h, W2c[...], preferred_element_type=_F32) + b2c[...]
    uvec = y[:, :_OUT]
    ivec = y[:, _OUT:]
    un = jnp.sqrt(jnp.sum(uvec * uvec, axis=1))
    inrm = jnp.sqrt(jnp.sum(ivec * ivec, axis=1))
    denom = jnp.maximum(un, 1e-12) * jnp.maximum(inrm, 1e-12)
    out[...] = jnp.sum(uvec * ivec, axis=1) / denom


def _row_spec(cols):
    return pl.BlockSpec((_BT, cols), lambda i: (i, 0))


def _full_spec(r, c):
    return pl.BlockSpec((r, c), lambda i: (0, 0))


_tc_towers = pl.pallas_call(
    _tc_towers_body,
    grid=(_B // _BT,),
    in_specs=[
        _row_spec(_EMB),
        _row_spec(_EMB),
        _row_spec(128),
        _row_spec(128),
        _row_spec(128),
        pl.BlockSpec((_BT,), lambda i: (i,)),
        pl.BlockSpec((_BT,), lambda i: (i,)),
        _full_spec(_K1, _H2),
        _full_spec(1, _HID),
        _full_spec(1, _HID),
        _full_spec(1, _H2),
        _full_spec(_H2, 2 * _OUT),
        _full_spec(1, 2 * _OUT),
    ],
    out_specs=pl.BlockSpec((_BT,), lambda i: (i,)),
    out_shape=jax.ShapeDtypeStruct((_B,), _F32),
)


def kernel(user_idx, user_norm_price, item_idx, item_cat, item_color,
           item_graphic, item_norm_price, user_table, item_table, cat_table,
           color_table, graphic_table, uW1, ub1, uW2, ub2, iW1, ib1, iW2, ib2):
    i32 = jnp.int32
    pad = ((0, 0), (0, 128 - _TAG))
    ie = _sc_item()(item_idx.astype(i32), item_table)
    ue, ce, cle, ge = _sc_user_tags()(
        user_idx.astype(i32), item_cat.astype(i32),
        item_color.astype(i32), item_graphic.astype(i32),
        user_table,
        jnp.pad(cat_table, pad), jnp.pad(color_table, pad),
        jnp.pad(graphic_table, pad))
    W1c = jnp.zeros((_K1, _H2), _F32)
    W1c = W1c.at[:_EMB, :_HID].set(uW1[:_EMB])
    W1c = W1c.at[_EMB:, _HID:].set(iW1[:_K1 - _EMB])
    W2c = jnp.zeros((_H2, 2 * _OUT), _F32)
    W2c = W2c.at[:_HID, :_OUT].set(uW2)
    W2c = W2c.at[_HID:, _OUT:].set(iW2)
    b1c = jnp.concatenate([ub1, ib1])[None, :]
    b2c = jnp.concatenate([ub2, ib2])[None, :]
    return _tc_towers(
        ue, ie, ce, cle, ge, user_norm_price, item_norm_price,
        W1c, uW1[_EMB:], iW1[_K1 - _EMB:], b1c, W2c, b2c)
